# baseline (speedup 1.0000x reference)
import functools

import jax
import jax.numpy as jnp
import numpy as np
from jax import lax
from jax.experimental import pallas as pl
from jax.experimental.pallas import tpu as pltpu
from jax.experimental.pallas import tpu_sc as plsc

_N = 10000
_E = 320000
_D = 128
_HC = 64
_SW = 96

_NC = 2
_NS = 16
_NW = _NC * _NS
_CHUNK = 128
_CPT = 80
_EPAD = _NW * _CPT * _CHUNK
_ACC_ROWS = 10112
_RPT = _ACC_ROWS // _NS


def _stage_chan(p):
    v, l = divmod(p, 16)
    return 8 * (4 * (v // 2) + l // 4) + 4 * (v % 2) + l % 4


_CPT0 = 64
_CPT1 = 96


def _shuf(v, idx):
    dnums = lax.GatherDimensionNumbers(
        offset_dims=(), collapsed_slice_dims=(0,), start_index_map=(0,))
    return lax.gather(v, idx[:, None], dnums, (1,),
                      mode=lax.GatherScatterMode.PROMISE_IN_BOUNDS)


def _project(x, w1, w2):
    def body(x_ref, w1_ref, w2_ref, o1_ref, o2_ref):
        xb = x_ref[...]
        o1_ref[...] = jnp.dot(xb, w1_ref[...],
                              preferred_element_type=jnp.float32)
        o2_ref[...] = jnp.dot(xb, w2_ref[...],
                              preferred_element_type=jnp.float32)

    return pl.pallas_call(
        body,
        grid=(5,),
        in_specs=[
            pl.BlockSpec((2000, _D), lambda i: (i, 0)),
            pl.BlockSpec((_D, _HC), lambda i: (0, 0)),
            pl.BlockSpec((_D, _HC), lambda i: (0, 0)),
        ],
        out_specs=[
            pl.BlockSpec((2000, _HC), lambda i: (i, 0)),
            pl.BlockSpec((2000, _HC), lambda i: (i, 0)),
        ],
        out_shape=[
            jax.ShapeDtypeStruct((_N, _HC), jnp.float32),
            jax.ShapeDtypeStruct((_N, _HC), jnp.float32),
        ],
    )(x, w1, w2)


def _sc_edge_pass(p1, p2, src, dst, a_perm, zeros):
    mesh = plsc.VectorSubcoreMesh(core_axis_name="c", subcore_axis_name="s")

    @functools.partial(
        pl.kernel,
        mesh=mesh,
        compiler_params=pltpu.CompilerParams(use_tc_tiling_on_sc=False),
        out_type=jax.ShapeDtypeStruct((_NC, _ACC_ROWS, _SW), jnp.float32),
        scratch_types=[
            pltpu.VMEM((2, _CHUNK), jnp.int32),
            pltpu.VMEM((2, _CHUNK), jnp.int32),
            pltpu.VMEM((2, _CHUNK, _HC), jnp.float32),
            pltpu.VMEM((2, _CHUNK, _HC), jnp.float32),
            pltpu.VMEM((_CHUNK, _SW), jnp.float32),
            pltpu.VMEM((_CHUNK,), jnp.int32),
            pltpu.VMEM((_HC,), jnp.float32),
            pltpu.VMEM_SHARED((_ACC_ROWS, _SW), jnp.float32),
            pltpu.SemaphoreType.DMA,
            pltpu.SemaphoreType.DMA,
            pltpu.SemaphoreType.DMA,
            pltpu.SemaphoreType.DMA,
        ],
    )
    def k(p1_hbm, p2_hbm, src_hbm, dst_hbm, a_hbm, z_hbm, out_hbm,
          sidx, didx, g1, g2, stage, dscat, a_v, acc,
          isem0, isem1, gsem0, gsem1):
        cid = lax.axis_index("c")
        sid = lax.axis_index("s")
        isem = [isem0, isem1]
        gsem = [gsem0, gsem1]
        cpt = jnp.where(cid == 0, _CPT0, _CPT1)
        coff = jnp.where(cid == 0, 0, _NS * _CPT0)

        def chunk_base(c):
            cc = jnp.minimum(c, cpt - 1)
            return (coff + sid * cpt + cc) * _CHUNK

        def issue_idx(c, b):
            base = chunk_base(c)
            pltpu.async_copy(src_hbm.at[pl.ds(base, _CHUNK)], sidx.at[b],
                             isem[b])
            pltpu.async_copy(dst_hbm.at[pl.ds(base, _CHUNK)], didx.at[b],
                             isem[b])

        def wait_idx(b):
            pltpu.make_async_copy(src_hbm.at[pl.ds(0, _CHUNK)], sidx.at[b],
                                  isem[b]).wait()
            pltpu.make_async_copy(dst_hbm.at[pl.ds(0, _CHUNK)], didx.at[b],
                                  isem[b]).wait()

        def issue_gathers(b):
            pltpu.async_copy(p1_hbm.at[sidx.at[b]], g1.at[b], gsem[b])
            pltpu.async_copy(p2_hbm.at[didx.at[b]], g2.at[b], gsem[b])

        def wait_gathers(b):
            pltpu.make_async_copy(p1_hbm.at[sidx.at[b]], g1.at[b],
                                  gsem[b]).wait()
            pltpu.make_async_copy(p2_hbm.at[didx.at[b]], g2.at[b],
                                  gsem[b]).wait()

        pltpu.sync_copy(z_hbm.at[pl.ds(sid * _RPT, _RPT)],
                        acc.at[pl.ds(sid * _RPT, _RPT)])
        pltpu.sync_copy(a_hbm, a_v)
        plsc.subcore_barrier()

        iot = lax.iota(jnp.int32, 16)
        x2 = jnp.bitwise_xor(iot, 2)
        x1 = jnp.bitwise_xor(iot, 1)
        a_g = [a_v[pl.ds(16 * g, 16)] for g in range(4)]

        issue_idx(0, 0)
        issue_idx(1, 1)
        wait_idx(0)
        issue_gathers(0)

        def pair_body(tp, carry):
            for b in range(2):
                c = 2 * tp + b
                wait_idx(1 - b)
                issue_gathers(1 - b)
                wait_gathers(b)
                for tcopy in range(_CHUNK // 16):
                    dscat[pl.ds(16 * tcopy, 16)] = didx[b, pl.ds(16 * tcopy, 16)]
                issue_idx(c + 2, b)

                @plsc.parallel_loop(0, _CHUNK, unroll=8)
                def edge(j):
                    for pair in range(2):
                        ra1 = g1[b, j, pl.ds(32 * pair, 16)]
                        rb1 = g1[b, j, pl.ds(32 * pair + 16, 16)]
                        ua = ra1 + g2[b, j, pl.ds(32 * pair, 16)]
                        ub = rb1 + g2[b, j, pl.ds(32 * pair + 16, 16)]
                        ta = jnp.maximum(ua, 0.3 * ua) * a_g[2 * pair]
                        tb = jnp.maximum(ub, 0.3 * ub) * a_g[2 * pair + 1]
                        t = ta + tb
                        t = t + _shuf(t, x2)
                        t = t + _shuf(t, x1)
                        w = jnp.exp(t)
                        stage[j, pl.ds(32 * pair, 16)] = ra1 * w
                        stage[j, pl.ds(32 * pair + 16, 16)] = rb1 * w
                        stage[j, pl.ds(_HC + 16 * pair, 16)] = w
                pltpu.sync_copy(stage, acc.at[dscat], add=True)
            return carry

        lax.fori_loop(0, cpt // 2, pair_body, 0)
        wait_idx(1)
        wait_gathers(0)
        plsc.subcore_barrier()
        pltpu.sync_copy(acc.at[pl.ds(sid * _RPT, _RPT)],
                        out_hbm.at[cid, pl.ds(sid * _RPT, _RPT)])

    return k(p1, p2, src, dst, a_perm, zeros)


def _combine(parts):
    def body(p_ref, o_ref):
        s = p_ref[0] + p_ref[1]
        num = s[:, :_HC]
        d0 = s[:, _HC:_HC + 16]
        d1 = s[:, _HC + 16:]
        den = jnp.concatenate([d0, d0, d1, d1], axis=1)
        o_ref[...] = jnp.where(den > 0.0, num / den, 0.0)

    return pl.pallas_call(
        body,
        grid=(8,),
        in_specs=[
            pl.BlockSpec((2, _ACC_ROWS // 8, _SW), lambda i: (0, i, 0)),
        ],
        out_specs=pl.BlockSpec((_ACC_ROWS // 8, _HC), lambda i: (i, 0)),
        out_shape=jax.ShapeDtypeStruct((_ACC_ROWS, _HC), jnp.float32),
    )(parts)


def kernel(x, edge_index, w1, w2, a):
    src = edge_index[0].astype(jnp.int32)
    dst = edge_index[1].astype(jnp.int32)
    npad = _EPAD - _E
    src = jnp.concatenate([src, jnp.zeros((npad,), jnp.int32)])
    junk = _N + jnp.arange(npad, dtype=jnp.int32) % (_ACC_ROWS - _N)
    dst = jnp.concatenate([dst, junk])
    scols = np.array([_stage_chan(p) for p in range(_HC)], np.int32)
    inv = np.empty(_HC, np.int32)
    inv[scols] = np.arange(_HC, dtype=np.int32)
    jcols = jnp.asarray(scols)
    p1, p2 = _project(x, w1[:, jcols], w2[:, jcols])
    zeros = jnp.zeros((_ACC_ROWS, _SW), jnp.float32)
    a_perm = a.reshape(_HC)[jnp.asarray(scols)]
    parts = _sc_edge_pass(p1, p2, src, dst, a_perm, zeros)
    return _combine(parts)[:_N, jnp.asarray(inv)]


# speedup vs baseline: 1.0199x; 1.0199x over previous
import functools

import jax
import jax.numpy as jnp
import numpy as np
from jax import lax
from jax.experimental import pallas as pl
from jax.experimental.pallas import tpu as pltpu
from jax.experimental.pallas import tpu_sc as plsc

_N = 10000
_E = 320000
_D = 128
_HC = 64
_SW = 96

_NC = 2
_NS = 16
_NW = _NC * _NS
_CHUNK = 128
_CPT = 80
_EPAD = _NW * _CPT * _CHUNK
_ACC_ROWS = 10112
_RPT = _ACC_ROWS // _NS


def _stage_chan(p):
    v, l = divmod(p, 16)
    return 8 * (4 * (v // 2) + l // 4) + 4 * (v % 2) + l % 4


_CPT0 = 64
_CPT1 = 96


def _shuf(v, idx):
    dnums = lax.GatherDimensionNumbers(
        offset_dims=(), collapsed_slice_dims=(0,), start_index_map=(0,))
    return lax.gather(v, idx[:, None], dnums, (1,),
                      mode=lax.GatherScatterMode.PROMISE_IN_BOUNDS)


def _project(x, w1, w2):
    def body(x_ref, w1_ref, w2_ref, o1_ref, o2_ref):
        xb = x_ref[...]
        o1_ref[...] = jnp.dot(xb, w1_ref[...],
                              preferred_element_type=jnp.float32)
        o2_ref[...] = jnp.dot(xb, w2_ref[...],
                              preferred_element_type=jnp.float32)

    return pl.pallas_call(
        body,
        grid=(5,),
        in_specs=[
            pl.BlockSpec((2000, _D), lambda i: (i, 0)),
            pl.BlockSpec((_D, _HC), lambda i: (0, 0)),
            pl.BlockSpec((_D, _HC), lambda i: (0, 0)),
        ],
        out_specs=[
            pl.BlockSpec((2000, _HC), lambda i: (i, 0)),
            pl.BlockSpec((2000, _HC), lambda i: (i, 0)),
        ],
        out_shape=[
            jax.ShapeDtypeStruct((_N, _HC), jnp.float32),
            jax.ShapeDtypeStruct((_N, _HC), jnp.float32),
        ],
    )(x, w1, w2)


def _sc_edge_pass(p1, p2, src, dst, a_perm, zeros):
    mesh = plsc.VectorSubcoreMesh(core_axis_name="c", subcore_axis_name="s")

    @functools.partial(
        pl.kernel,
        mesh=mesh,
        compiler_params=pltpu.CompilerParams(use_tc_tiling_on_sc=False),
        out_type=jax.ShapeDtypeStruct((_NC, _ACC_ROWS, _SW), jnp.float32),
        scratch_types=[
            pltpu.VMEM((2, _CHUNK), jnp.int32),
            pltpu.VMEM((2, _CHUNK), jnp.int32),
            pltpu.VMEM((2, _CHUNK, _HC), jnp.float32),
            pltpu.VMEM((2, _CHUNK, _HC), jnp.float32),
            pltpu.VMEM((_CHUNK, _SW), jnp.float32),
            pltpu.VMEM((_CHUNK,), jnp.int32),
            pltpu.VMEM((_HC,), jnp.float32),
            pltpu.VMEM_SHARED((_ACC_ROWS, _SW), jnp.float32),
            pltpu.SemaphoreType.DMA,
            pltpu.SemaphoreType.DMA,
            pltpu.SemaphoreType.DMA,
            pltpu.SemaphoreType.DMA,
        ],
    )
    def k(p1_hbm, p2_hbm, src_hbm, dst_hbm, a_hbm, z_hbm, out_hbm,
          sidx, didx, g1, g2, stage, dscat, a_v, acc,
          isem0, isem1, gsem0, gsem1):
        cid = lax.axis_index("c")
        sid = lax.axis_index("s")
        wid = cid * _NS + sid
        isem = [isem0, isem1]
        gsem = [gsem0, gsem1]

        def chunk_base(c):
            cc = jnp.minimum(c, _CPT - 1)
            return (wid * _CPT + cc) * _CHUNK

        def issue_idx(c, b):
            base = chunk_base(c)
            pltpu.async_copy(src_hbm.at[pl.ds(base, _CHUNK)], sidx.at[b],
                             isem[b])
            pltpu.async_copy(dst_hbm.at[pl.ds(base, _CHUNK)], didx.at[b],
                             isem[b])

        def wait_idx(b):
            pltpu.make_async_copy(src_hbm.at[pl.ds(0, _CHUNK)], sidx.at[b],
                                  isem[b]).wait()
            pltpu.make_async_copy(dst_hbm.at[pl.ds(0, _CHUNK)], didx.at[b],
                                  isem[b]).wait()

        def issue_gathers(b):
            pltpu.async_copy(p1_hbm.at[sidx.at[b]], g1.at[b], gsem[b])
            pltpu.async_copy(p2_hbm.at[didx.at[b]], g2.at[b], gsem[b])

        def wait_gathers(b):
            pltpu.make_async_copy(p1_hbm.at[sidx.at[b]], g1.at[b],
                                  gsem[b]).wait()
            pltpu.make_async_copy(p2_hbm.at[didx.at[b]], g2.at[b],
                                  gsem[b]).wait()

        pltpu.sync_copy(z_hbm.at[pl.ds(sid * _RPT, _RPT)],
                        acc.at[pl.ds(sid * _RPT, _RPT)])
        pltpu.sync_copy(a_hbm, a_v)
        plsc.subcore_barrier()

        iot = lax.iota(jnp.int32, 16)
        x2 = jnp.bitwise_xor(iot, 2)
        x1 = jnp.bitwise_xor(iot, 1)
        a_g = [a_v[pl.ds(16 * g, 16)] for g in range(4)]

        issue_idx(0, 0)
        issue_idx(1, 1)
        wait_idx(0)
        issue_gathers(0)

        def pair_body(tp, carry):
            for b in range(2):
                c = 2 * tp + b
                wait_idx(1 - b)
                issue_gathers(1 - b)
                wait_gathers(b)
                for tcopy in range(_CHUNK // 16):
                    dscat[pl.ds(16 * tcopy, 16)] = didx[b, pl.ds(16 * tcopy, 16)]
                issue_idx(c + 2, b)

                @plsc.parallel_loop(0, _CHUNK, unroll=8)
                def edge(j):
                    for pair in range(2):
                        ra1 = g1[b, j, pl.ds(32 * pair, 16)]
                        rb1 = g1[b, j, pl.ds(32 * pair + 16, 16)]
                        ua = ra1 + g2[b, j, pl.ds(32 * pair, 16)]
                        ub = rb1 + g2[b, j, pl.ds(32 * pair + 16, 16)]
                        ta = jnp.maximum(ua, 0.3 * ua) * a_g[2 * pair]
                        tb = jnp.maximum(ub, 0.3 * ub) * a_g[2 * pair + 1]
                        t = ta + tb
                        t = t + _shuf(t, x2)
                        t = t + _shuf(t, x1)
                        w = jnp.exp(t)
                        stage[j, pl.ds(32 * pair, 16)] = ra1 * w
                        stage[j, pl.ds(32 * pair + 16, 16)] = rb1 * w
                        stage[j, pl.ds(_HC + 16 * pair, 16)] = w
                pltpu.sync_copy(stage, acc.at[dscat], add=True)
            return carry

        lax.fori_loop(0, _CPT // 2, pair_body, 0)
        wait_idx(1)
        wait_gathers(0)
        plsc.subcore_barrier()
        pltpu.sync_copy(acc.at[pl.ds(sid * _RPT, _RPT)],
                        out_hbm.at[cid, pl.ds(sid * _RPT, _RPT)])

    return k(p1, p2, src, dst, a_perm, zeros)


def _combine(parts):
    def body(p_ref, o_ref):
        s = p_ref[0] + p_ref[1]
        num = s[:, :_HC]
        d0 = s[:, _HC:_HC + 16]
        d1 = s[:, _HC + 16:]
        den = jnp.concatenate([d0, d0, d1, d1], axis=1)
        o_ref[...] = jnp.where(den > 0.0, num / den, 0.0)

    return pl.pallas_call(
        body,
        grid=(8,),
        in_specs=[
            pl.BlockSpec((2, _ACC_ROWS // 8, _SW), lambda i: (0, i, 0)),
        ],
        out_specs=pl.BlockSpec((_ACC_ROWS // 8, _HC), lambda i: (i, 0)),
        out_shape=jax.ShapeDtypeStruct((_ACC_ROWS, _HC), jnp.float32),
    )(parts)


def kernel(x, edge_index, w1, w2, a):
    src = edge_index[0].astype(jnp.int32)
    dst = edge_index[1].astype(jnp.int32)
    npad = _EPAD - _E
    src = jnp.concatenate([src, jnp.zeros((npad,), jnp.int32)])
    junk = _N + jnp.arange(npad, dtype=jnp.int32) % (_ACC_ROWS - _N)
    dst = jnp.concatenate([dst, junk])
    scols = np.array([_stage_chan(p) for p in range(_HC)], np.int32)
    inv = np.empty(_HC, np.int32)
    inv[scols] = np.arange(_HC, dtype=np.int32)
    jcols = jnp.asarray(scols)
    p1, p2 = _project(x, w1[:, jcols], w2[:, jcols])
    zeros = jnp.zeros((_ACC_ROWS, _SW), jnp.float32)
    a_perm = a.reshape(_HC)[jnp.asarray(scols)]
    parts = _sc_edge_pass(p1, p2, src, dst, a_perm, zeros)
    return _combine(parts)[:_N, jnp.asarray(inv)]


# speedup vs baseline: 1.8039x; 1.7686x over previous
import functools

import jax
import jax.numpy as jnp
import numpy as np
from jax import lax
from jax.experimental import pallas as pl
from jax.experimental.pallas import tpu as pltpu
from jax.experimental.pallas import tpu_sc as plsc

_N = 10000
_E = 320000
_D = 128
_HC = 64
_SW = 96

_NC = 2
_NS = 16
_NW = _NC * _NS
_CHUNK = 128
_CPT = 80
_EPAD = _NW * _CPT * _CHUNK
_ACC_ROWS = 10112
_RPT = _ACC_ROWS // _NS


def _stage_chan(p):
    v, l = divmod(p, 16)
    return 8 * (4 * (v // 2) + l // 4) + 4 * (v % 2) + l % 4


_CPT0 = 64
_CPT1 = 96


def _shuf(v, idx):
    dnums = lax.GatherDimensionNumbers(
        offset_dims=(), collapsed_slice_dims=(0,), start_index_map=(0,))
    return lax.gather(v, idx[:, None], dnums, (1,),
                      mode=lax.GatherScatterMode.PROMISE_IN_BOUNDS)


def _project(x, w1, w2):
    def body(x_ref, w1_ref, w2_ref, o1_ref, o2_ref):
        xb = x_ref[...]
        o1_ref[...] = jnp.dot(xb, w1_ref[...],
                              preferred_element_type=jnp.float32)
        o2_ref[...] = jnp.dot(xb, w2_ref[...],
                              preferred_element_type=jnp.float32)

    return pl.pallas_call(
        body,
        grid=(5,),
        in_specs=[
            pl.BlockSpec((2000, _D), lambda i: (i, 0)),
            pl.BlockSpec((_D, _HC), lambda i: (0, 0)),
            pl.BlockSpec((_D, _HC), lambda i: (0, 0)),
        ],
        out_specs=[
            pl.BlockSpec((2000, _HC), lambda i: (i, 0)),
            pl.BlockSpec((2000, _HC), lambda i: (i, 0)),
        ],
        out_shape=[
            jax.ShapeDtypeStruct((_N, _HC), jnp.float32),
            jax.ShapeDtypeStruct((_N, _HC), jnp.float32),
        ],
    )(x, w1, w2)


def _sc_edge_pass(p1, p2, src, dst, a_perm, zeros):
    mesh = plsc.VectorSubcoreMesh(core_axis_name="c", subcore_axis_name="s")

    @functools.partial(
        pl.kernel,
        mesh=mesh,
        compiler_params=pltpu.CompilerParams(use_tc_tiling_on_sc=False),
        out_type=jax.ShapeDtypeStruct((_NC, _ACC_ROWS, _SW), jnp.float32),
        scratch_types=[
            pltpu.VMEM((2, _CHUNK), jnp.int32),
            pltpu.VMEM((2, _CHUNK), jnp.int32),
            pltpu.VMEM((2, _CHUNK, _HC), jnp.float32),
            pltpu.VMEM((2, _CHUNK, _HC), jnp.float32),
            pltpu.VMEM((_CHUNK, _SW), jnp.float32),
            pltpu.VMEM((_CHUNK,), jnp.int32),
            pltpu.VMEM((_HC,), jnp.float32),
            pltpu.VMEM_SHARED((_ACC_ROWS, _SW), jnp.float32),
            pltpu.SemaphoreType.DMA,
            pltpu.SemaphoreType.DMA,
            pltpu.SemaphoreType.DMA,
            pltpu.SemaphoreType.DMA,
        ],
    )
    def k(p1_hbm, p2_hbm, src_hbm, dst_hbm, a_hbm, z_hbm, out_hbm,
          sidx, didx, g1, g2, stage, dscat, a_v, acc,
          isem0, isem1, gsem0, gsem1):
        cid = lax.axis_index("c")
        sid = lax.axis_index("s")
        wid = cid * _NS + sid
        isem = [isem0, isem1]
        gsem = [gsem0, gsem1]

        def chunk_base(c):
            cc = jnp.minimum(c, _CPT - 1)
            return (wid * _CPT + cc) * _CHUNK

        def issue_idx(c, b):
            base = chunk_base(c)
            pltpu.async_copy(src_hbm.at[pl.ds(base, _CHUNK)], sidx.at[b],
                             isem[b])
            pltpu.async_copy(dst_hbm.at[pl.ds(base, _CHUNK)], didx.at[b],
                             isem[b])

        def wait_idx(b):
            pltpu.make_async_copy(src_hbm.at[pl.ds(0, _CHUNK)], sidx.at[b],
                                  isem[b]).wait()
            pltpu.make_async_copy(dst_hbm.at[pl.ds(0, _CHUNK)], didx.at[b],
                                  isem[b]).wait()

        def issue_gathers(b):
            pltpu.async_copy(p1_hbm.at[sidx.at[b]], g1.at[b], gsem[b])
            pltpu.async_copy(p2_hbm.at[didx.at[b]], g2.at[b], gsem[b])

        def wait_gathers(b):
            pltpu.make_async_copy(p1_hbm.at[sidx.at[b]], g1.at[b],
                                  gsem[b]).wait()
            pltpu.make_async_copy(p2_hbm.at[didx.at[b]], g2.at[b],
                                  gsem[b]).wait()

        pltpu.sync_copy(z_hbm.at[pl.ds(sid * _RPT, _RPT)],
                        acc.at[pl.ds(sid * _RPT, _RPT)])
        pltpu.sync_copy(a_hbm, a_v)
        plsc.subcore_barrier()

        iot = lax.iota(jnp.int32, 16)
        x2 = jnp.bitwise_xor(iot, 2)
        x1 = jnp.bitwise_xor(iot, 1)
        a_g = [a_v[pl.ds(16 * g, 16)] for g in range(4)]

        issue_idx(0, 0)
        issue_idx(1, 1)
        wait_idx(0)
        issue_gathers(0)

        def pair_body(tp, carry):
            for b in range(2):
                c = 2 * tp + b
                wait_idx(1 - b)
                issue_gathers(1 - b)
                wait_gathers(b)
                for tcopy in range(_CHUNK // 16):
                    dscat[pl.ds(16 * tcopy, 16)] = didx[b, pl.ds(16 * tcopy, 16)]
                issue_idx(c + 2, b)

                @plsc.parallel_loop(0, _CHUNK, unroll=8)
                def edge(j):
                    for pair in range(2):
                        ra1 = g1[b, j, pl.ds(32 * pair, 16)]
                        rb1 = g1[b, j, pl.ds(32 * pair + 16, 16)]
                        ua = ra1 + g2[b, j, pl.ds(32 * pair, 16)]
                        ub = rb1 + g2[b, j, pl.ds(32 * pair + 16, 16)]
                        ta = jnp.maximum(ua, 0.3 * ua) * a_g[2 * pair]
                        tb = jnp.maximum(ub, 0.3 * ub) * a_g[2 * pair + 1]
                        t = ta + tb
                        t = t + _shuf(t, x2)
                        t = t + _shuf(t, x1)
                        w = jnp.exp(t)
                        stage[j, pl.ds(32 * pair, 16)] = ra1 * w
                        stage[j, pl.ds(32 * pair + 16, 16)] = rb1 * w
                        stage[j, pl.ds(_HC + 16 * pair, 16)] = w
                pltpu.sync_copy(stage, acc.at[dscat], add=True)
            return carry

        lax.fori_loop(0, _CPT // 2, pair_body, 0)
        wait_idx(1)
        wait_gathers(0)
        plsc.subcore_barrier()
        pltpu.sync_copy(acc.at[pl.ds(sid * _RPT, _RPT)],
                        out_hbm.at[cid, pl.ds(sid * _RPT, _RPT)])

    return k(p1, p2, src, dst, a_perm, zeros)


def _perm_mats():
    m = np.zeros((_HC, _HC), np.float32)
    for p in range(_HC):
        m[p, _stage_chan(p)] = 1.0
    md = np.zeros((32, _HC), np.float32)
    for c in range(_HC):
        h = c // 8
        md[16 * (h // 4) + 4 * (h % 4), c] = 1.0
    return jnp.asarray(m), jnp.asarray(md)


def _combine(parts, m_num, m_den):
    def body(p_ref, mn_ref, md_ref, o_ref):
        s = p_ref[0] + p_ref[1]
        num = jnp.dot(s[:, :_HC], mn_ref[...],
                      preferred_element_type=jnp.float32,
                      precision=lax.Precision.HIGHEST)
        den = jnp.dot(s[:, _HC:], md_ref[...],
                      preferred_element_type=jnp.float32,
                      precision=lax.Precision.HIGHEST)
        o_ref[...] = jnp.where(den > 0.0, num / den, 0.0)

    return pl.pallas_call(
        body,
        grid=(8,),
        in_specs=[
            pl.BlockSpec((2, _ACC_ROWS // 8, _SW), lambda i: (0, i, 0)),
            pl.BlockSpec((_HC, _HC), lambda i: (0, 0)),
            pl.BlockSpec((32, _HC), lambda i: (0, 0)),
        ],
        out_specs=pl.BlockSpec((_ACC_ROWS // 8, _HC), lambda i: (i, 0)),
        out_shape=jax.ShapeDtypeStruct((_ACC_ROWS, _HC), jnp.float32),
    )(parts, m_num, m_den)


def kernel(x, edge_index, w1, w2, a):
    src = edge_index[0].astype(jnp.int32)
    dst = edge_index[1].astype(jnp.int32)
    npad = _EPAD - _E
    src = jnp.concatenate([src, jnp.zeros((npad,), jnp.int32)])
    junk = _N + jnp.arange(npad, dtype=jnp.int32) % (_ACC_ROWS - _N)
    dst = jnp.concatenate([dst, junk])
    jcols = jnp.asarray(np.array([_stage_chan(p) for p in range(_HC)],
                                 np.int32))
    p1, p2 = _project(x, w1[:, jcols], w2[:, jcols])
    zeros = jnp.zeros((_ACC_ROWS, _SW), jnp.float32)
    a_perm = a.reshape(_HC)[jcols]
    parts = _sc_edge_pass(p1, p2, src, dst, a_perm, zeros)
    m_num, m_den = _perm_mats()
    return _combine(parts, m_num, m_den)[:_N]
